# baseline (speedup 1.0000x reference)
import dataclasses
import functools

import jax
import jax.numpy as jnp
from jax import lax
from jax.experimental import pallas as pl
from jax.experimental.pallas import tpu as pltpu
from jax.experimental.pallas import tpu_sc as plsc

B = 16384
D = 64
V = 1000000
L = 16
NC, NS = 2, 16
NW = NC * NS
BPW = B // NW
C = 32
NCH = BPW // C
NSTR = C * D // 128

_mesh = plsc.VectorSubcoreMesh(core_axis_name="c", subcore_axis_name="s")

_cp = pltpu.CompilerParams(use_tc_tiling_on_sc=False)
if "needs_layout_passes" in pltpu.CompilerParams.__dataclass_fields__:
    _cp = dataclasses.replace(_cp, needs_layout_passes=False)


@functools.partial(
    pl.kernel,
    out_type=(
        jax.ShapeDtypeStruct((B, L), jnp.float32),
        jax.ShapeDtypeStruct((NW, L), jnp.float32),
    ),
    mesh=_mesh,
    compiler_params=_cp,
    scratch_types=[
        pltpu.VMEM((BPW,), jnp.int32),
        pltpu.VMEM((BPW,), jnp.int32),
        pltpu.VMEM((BPW,), jnp.int32),
        pltpu.VMEM((2, NSTR, 128), jnp.int32),
        pltpu.VMEM((2, NSTR, 128), jnp.int32),
        pltpu.VMEM((2, NSTR, 128), jnp.int32),
        pltpu.VMEM((2, C * D), jnp.float32),
        pltpu.VMEM((2, C * D), jnp.float32),
        pltpu.VMEM((2, C * D), jnp.float32),
        pltpu.VMEM((BPW, L), jnp.float32),
        pltpu.VMEM((L,), jnp.float32),
        pltpu.SemaphoreType.DMA,
        pltpu.SemaphoreType.DMA,
    ],
)
def _bpr_sc(uid_hbm, pid_hbm, nid_hbm, wt_hbm, ht_hbm, d_hbm, reg_hbm,
            uid_v, pid_v, nid_v, iu_v, ip_v, in_v, u_v, p_v, n_v,
            d_v, racc_v, sem0, sem1):
    wid = lax.axis_index("s") * NC + lax.axis_index("c")
    pltpu.sync_copy(uid_hbm.at[wid], uid_v)
    pltpu.sync_copy(pid_hbm.at[wid], pid_v)
    pltpu.sync_copy(nid_hbm.at[wid], nid_v)

    jc = [(lax.iota(jnp.int32, L) + c * L) * V for c in range(D // L)]

    def build_idx(k, buf):
        for half in range(C // L):
            r0 = half * L
            uvec = uid_v[pl.ds(k * C + r0, L)]
            pvec = pid_v[pl.ds(k * C + r0, L)]
            nvec = nid_v[pl.ds(k * C + r0, L)]
            for r in range(L):
                row = r0 + r
                s, off = divmod(row * D, 128)
                for idx_v, vec in ((iu_v, uvec), (ip_v, pvec), (in_v, nvec)):
                    base = jnp.full((L,), vec[r], jnp.int32)
                    for c in range(D // L):
                        o = off + c * L
                        idx_v[buf, s + o // 128, pl.ds(o % 128, L)] = jc[c] + base

    def fire(k, buf, sem):
        for st in range(NSTR):
            dst = pl.ds(st * 128, 128)
            pltpu.async_copy(wt_hbm.at[iu_v.at[buf].at[st]], u_v.at[buf].at[dst], sem)
            pltpu.async_copy(ht_hbm.at[ip_v.at[buf].at[st]], p_v.at[buf].at[dst], sem)
            pltpu.async_copy(ht_hbm.at[in_v.at[buf].at[st]], n_v.at[buf].at[dst], sem)

    def drain(buf, sem):
        pltpu.make_async_copy(wt_hbm.at[pl.ds(0, C * D)], u_v.at[buf], sem).wait()
        pltpu.make_async_copy(ht_hbm.at[pl.ds(0, C * D)], p_v.at[buf], sem).wait()
        pltpu.make_async_copy(ht_hbm.at[pl.ds(0, C * D)], n_v.at[buf], sem).wait()

    def compute(k, buf):
        @pl.loop(0, C)
        def _(j):
            dv = None
            rs = None
            for c in range(D // L):
                sl = pl.ds(j * D + c * L, L)
                u = u_v[buf, sl]
                p = p_v[buf, sl]
                n = n_v[buf, sl]
                contrib = u * (p - n)
                sq = u * u + p * p + n * n
                dv = contrib if dv is None else dv + contrib
                rs = sq if rs is None else rs + sq
            d_v[k * C + j, :] = dv
            racc_v[...] = racc_v[...] + rs

    racc_v[...] = jnp.zeros((L,), jnp.float32)
    build_idx(0, 0)
    fire(0, 0, sem0)
    build_idx(1, 1)
    fire(1, 1, sem1)

    @pl.loop(0, NCH, step=2)
    def _(k):
        drain(0, sem0)
        compute(k, 0)

        @pl.when(k + 2 < NCH)
        def _():
            build_idx(k + 2, 0)
            fire(k + 2, 0, sem0)

        drain(1, sem1)
        compute(k + 1, 1)

        @pl.when(k + 3 < NCH)
        def _():
            build_idx(k + 3, 1)
            fire(k + 3, 1, sem1)

    pltpu.sync_copy(d_v, d_hbm.at[pl.ds(wid * BPW, BPW)])
    pltpu.sync_copy(racc_v, reg_hbm.at[wid])


def _finish_body(d_ref, reg_ref, o_ref):
    s = jnp.sum(d_ref[...], axis=1, keepdims=True)
    bpr = -jnp.sum(jax.nn.log_sigmoid(s))
    reg = 0.01 * jnp.sum(reg_ref[...])
    o_ref[...] = jnp.reshape(bpr + reg, (1, 1))


_finish = pl.pallas_call(
    _finish_body,
    out_shape=jax.ShapeDtypeStruct((1, 1), jnp.float32),
)


def kernel(data, W, H):
    uid = data[:, 0].reshape(NW, BPW)
    pid = data[:, 1].reshape(NW, BPW)
    nid = data[:, 2].reshape(NW, BPW)
    wt = W.T.reshape(D * V)
    ht = H.T.reshape(D * V)
    d_part, reg_part = _bpr_sc(uid, pid, nid, wt, ht)
    return _finish(d_part, reg_part)[0, 0]


# speedup vs baseline: 8.8909x; 8.8909x over previous
import functools

import jax
import jax.numpy as jnp
from jax import lax
from jax.experimental import pallas as pl
from jax.experimental.pallas import tpu as pltpu
from jax.experimental.pallas import tpu_sc as plsc

B = 16384
D = 64
V = 1000000
L = 16
NC, NS = 2, 16
NW = NC * NS
BPW = B // NW
C = 32
NCH = BPW // C

_mesh = plsc.VectorSubcoreMesh(core_axis_name="c", subcore_axis_name="s")


@functools.partial(
    pl.kernel,
    out_type=(
        jax.ShapeDtypeStruct((B, L), jnp.float32),
        jax.ShapeDtypeStruct((NW, L), jnp.float32),
    ),
    mesh=_mesh,
    scratch_types=[
        pltpu.VMEM((BPW,), jnp.int32),
        pltpu.VMEM((BPW,), jnp.int32),
        pltpu.VMEM((BPW,), jnp.int32),
        pltpu.VMEM((2, C, 2 * D), jnp.float32),
        pltpu.VMEM((2, C, 2 * D), jnp.float32),
        pltpu.VMEM((2, C, 2 * D), jnp.float32),
        pltpu.VMEM((BPW, L), jnp.float32),
        pltpu.VMEM((L,), jnp.float32),
        pltpu.SemaphoreType.DMA,
        pltpu.SemaphoreType.DMA,
    ],
)
def _bpr_sc(uid_hbm, pid_hbm, nid_hbm, w2_hbm, h2_hbm, d_hbm, reg_hbm,
            uid_v, pid_v, nid_v, u_v, p_v, n_v, d_v, racc_v, sem0, sem1):
    wid = lax.axis_index("s") * NC + lax.axis_index("c")
    pltpu.sync_copy(uid_hbm.at[wid], uid_v)
    pltpu.sync_copy(pid_hbm.at[wid], pid_v)
    pltpu.sync_copy(nid_hbm.at[wid], nid_v)

    def fire(k, buf, sem):
        @pl.loop(0, C, step=16)
        def _(j0):
            uvec = uid_v[pl.ds(k * C + j0, 16)]
            pvec = pid_v[pl.ds(k * C + j0, 16)]
            nvec = nid_v[pl.ds(k * C + j0, 16)]
            ul = lax.shift_right_logical(uvec, 1)
            pl_ = lax.shift_right_logical(pvec, 1)
            nl = lax.shift_right_logical(nvec, 1)
            for j in range(16):
                dst = pl.ds(j0 + j, 1)
                pltpu.async_copy(w2_hbm.at[pl.ds(ul[j], 1)], u_v.at[buf].at[dst], sem)
                pltpu.async_copy(h2_hbm.at[pl.ds(pl_[j], 1)], p_v.at[buf].at[dst], sem)
                pltpu.async_copy(h2_hbm.at[pl.ds(nl[j], 1)], n_v.at[buf].at[dst], sem)

    def drain(buf, sem):
        pltpu.make_async_copy(w2_hbm.at[pl.ds(0, C)], u_v.at[buf], sem).wait()
        pltpu.make_async_copy(h2_hbm.at[pl.ds(0, C)], p_v.at[buf], sem).wait()
        pltpu.make_async_copy(h2_hbm.at[pl.ds(0, C)], n_v.at[buf], sem).wait()

    def compute(k, buf):
        @pl.loop(0, C, step=16)
        def _(j0):
            ssl = pl.ds(k * C + j0, 16)
            um = lax.bitwise_and(uid_v[ssl], 1)
            pm = lax.bitwise_and(pid_v[ssl], 1)
            nm = lax.bitwise_and(nid_v[ssl], 1)
            for j in range(16):
                uf = jnp.full((L,), um[j], jnp.int32).astype(jnp.float32)
                pf = jnp.full((L,), pm[j], jnp.int32).astype(jnp.float32)
                nf = jnp.full((L,), nm[j], jnp.int32).astype(jnp.float32)
                dv = None
                rs = None
                for c in range(D // L):
                    lo = pl.ds(c * L, L)
                    hi = pl.ds(D + c * L, L)
                    ulo = u_v[buf, j0 + j, lo]
                    plo = p_v[buf, j0 + j, lo]
                    nlo = n_v[buf, j0 + j, lo]
                    u = ulo + uf * (u_v[buf, j0 + j, hi] - ulo)
                    p = plo + pf * (p_v[buf, j0 + j, hi] - plo)
                    n = nlo + nf * (n_v[buf, j0 + j, hi] - nlo)
                    contrib = u * (p - n)
                    sq = u * u + p * p + n * n
                    dv = contrib if dv is None else dv + contrib
                    rs = sq if rs is None else rs + sq
                d_v[k * C + j0 + j, :] = dv
                racc_v[...] = racc_v[...] + rs

    racc_v[...] = jnp.zeros((L,), jnp.float32)
    fire(0, 0, sem0)
    fire(1, 1, sem1)

    @pl.loop(0, NCH, step=2)
    def _(k):
        drain(0, sem0)
        compute(k, 0)

        @pl.when(k + 2 < NCH)
        def _():
            fire(k + 2, 0, sem0)

        drain(1, sem1)
        compute(k + 1, 1)

        @pl.when(k + 3 < NCH)
        def _():
            fire(k + 3, 1, sem1)

    pltpu.sync_copy(d_v, d_hbm.at[pl.ds(wid * BPW, BPW)])
    pltpu.sync_copy(racc_v, reg_hbm.at[wid])


def _finish_body(d_ref, reg_ref, o_ref):
    s = jnp.sum(d_ref[...], axis=1, keepdims=True)
    bpr = -jnp.sum(jax.nn.log_sigmoid(s))
    reg = 0.01 * jnp.sum(reg_ref[...])
    o_ref[...] = jnp.reshape(bpr + reg, (1, 1))


_finish = pl.pallas_call(
    _finish_body,
    out_shape=jax.ShapeDtypeStruct((1, 1), jnp.float32),
)


def kernel(data, W, H):
    uid = data[:, 0].reshape(NW, BPW)
    pid = data[:, 1].reshape(NW, BPW)
    nid = data[:, 2].reshape(NW, BPW)
    w2 = W.reshape(V // 2, 2 * D)
    h2 = H.reshape(V // 2, 2 * D)
    d_part, reg_part = _bpr_sc(uid, pid, nid, w2, h2)
    return _finish(d_part, reg_part)[0, 0]


# speedup vs baseline: 14.0646x; 1.5819x over previous
import functools

import jax
import jax.numpy as jnp
from jax import lax
from jax.experimental import pallas as pl
from jax.experimental.pallas import tpu as pltpu
from jax.experimental.pallas import tpu_sc as plsc

B = 16384
D = 64
L = 16
NC, NS = 2, 16
NW = NC * NS
BPW = B // NW
C = 64
NCH = BPW // C

_mesh = plsc.VectorSubcoreMesh(core_axis_name="c", subcore_axis_name="s")


@functools.partial(
    pl.kernel,
    out_type=(
        jax.ShapeDtypeStruct((B, L), jnp.float32),
        jax.ShapeDtypeStruct((NW, L), jnp.float32),
    ),
    mesh=_mesh,
    scratch_types=[
        pltpu.VMEM((BPW,), jnp.int32),
        pltpu.VMEM((BPW,), jnp.int32),
        pltpu.VMEM((BPW,), jnp.int32),
        pltpu.VMEM((2, C, D), jnp.float32),
        pltpu.VMEM((2, C, D), jnp.float32),
        pltpu.VMEM((2, C, D), jnp.float32),
        pltpu.VMEM((BPW, L), jnp.float32),
        pltpu.VMEM((L,), jnp.float32),
        pltpu.SemaphoreType.DMA,
        pltpu.SemaphoreType.DMA,
    ],
)
def _bpr_sc(uid_hbm, pid_hbm, nid_hbm, w_hbm, h_hbm, d_hbm, reg_hbm,
            uid_v, pid_v, nid_v, u_v, p_v, n_v, d_v, racc_v, sem0, sem1):
    wid = lax.axis_index("s") * NC + lax.axis_index("c")
    pltpu.sync_copy(uid_hbm.at[wid], uid_v)
    pltpu.sync_copy(pid_hbm.at[wid], pid_v)
    pltpu.sync_copy(nid_hbm.at[wid], nid_v)

    def fire(k, buf, sem):
        @pl.loop(0, C, step=16)
        def _(j0):
            uvec = uid_v[pl.ds(k * C + j0, 16)]
            pvec = pid_v[pl.ds(k * C + j0, 16)]
            nvec = nid_v[pl.ds(k * C + j0, 16)]
            for j in range(16):
                dst = pl.ds(j0 + j, 1)
                pltpu.async_copy(w_hbm.at[pl.ds(uvec[j], 1)], u_v.at[buf].at[dst], sem)
                pltpu.async_copy(h_hbm.at[pl.ds(pvec[j], 1)], p_v.at[buf].at[dst], sem)
                pltpu.async_copy(h_hbm.at[pl.ds(nvec[j], 1)], n_v.at[buf].at[dst], sem)

    def drain(buf, sem):
        pltpu.make_async_copy(w_hbm.at[pl.ds(0, C)], u_v.at[buf], sem).wait()
        pltpu.make_async_copy(h_hbm.at[pl.ds(0, C)], p_v.at[buf], sem).wait()
        pltpu.make_async_copy(h_hbm.at[pl.ds(0, C)], n_v.at[buf], sem).wait()

    def compute(k, buf):
        @pl.loop(0, C)
        def _(j):
            dv = None
            rs = None
            for c in range(D // L):
                sl = pl.ds(c * L, L)
                u = u_v[buf, j, sl]
                p = p_v[buf, j, sl]
                n = n_v[buf, j, sl]
                contrib = u * (p - n)
                sq = u * u + p * p + n * n
                dv = contrib if dv is None else dv + contrib
                rs = sq if rs is None else rs + sq
            d_v[k * C + j, :] = dv
            racc_v[...] = racc_v[...] + rs

    racc_v[...] = jnp.zeros((L,), jnp.float32)
    fire(0, 0, sem0)
    fire(1, 1, sem1)

    @pl.loop(0, NCH, step=2)
    def _(k):
        drain(0, sem0)
        compute(k, 0)

        @pl.when(k + 2 < NCH)
        def _():
            fire(k + 2, 0, sem0)

        drain(1, sem1)
        compute(k + 1, 1)

        @pl.when(k + 3 < NCH)
        def _():
            fire(k + 3, 1, sem1)

    pltpu.sync_copy(d_v, d_hbm.at[pl.ds(wid * BPW, BPW)])
    pltpu.sync_copy(racc_v, reg_hbm.at[wid])


def _finish_body(d_ref, reg_ref, o_ref):
    s = jnp.sum(d_ref[...], axis=1, keepdims=True)
    bpr = -jnp.sum(jax.nn.log_sigmoid(s))
    reg = 0.01 * jnp.sum(reg_ref[...])
    o_ref[...] = jnp.reshape(bpr + reg, (1, 1))


_finish = pl.pallas_call(
    _finish_body,
    out_shape=jax.ShapeDtypeStruct((1, 1), jnp.float32),
)


def kernel(data, W, H):
    uid = data[:, 0].reshape(NW, BPW)
    pid = data[:, 1].reshape(NW, BPW)
    nid = data[:, 2].reshape(NW, BPW)
    d_part, reg_part = _bpr_sc(uid, pid, nid, W, H)
    return _finish(d_part, reg_part)[0, 0]
